# baseline (speedup 1.0000x reference)
import jax
import jax.numpy as jnp
from jax import lax
from jax.experimental import pallas as pl
from jax.experimental.pallas import tpu as pltpu
from jax.experimental.pallas import tpu_sc as plsc

N = 10000
E = 320000
D = 128

NC, NS = 2, 16
NW = NC * NS
NPAD = 10240
EPW = E // NW
K = 80
NCH = EPW // K
ROWS_PT = NPAD // NS
APT = NPAD // NS

_mesh = plsc.VectorSubcoreMesh(
    core_axis_name="c", subcore_axis_name="s", num_cores=NC, num_subcores=NS)


def _hist_body(row_hbm, ones_hbm, zeros_hbm, out_hbm, ridx0, ridx1, ones_v,
               deg_s, is0, is1, ss0, ss1):
    ridxs = (ridx0, ridx1)
    isems = (is0, is1)
    ssems = (ss0, ss1)
    c = lax.axis_index("c")
    s = lax.axis_index("s")
    w = c * NS + s

    pltpu.sync_copy(zeros_hbm, deg_s.at[pl.ds(s * (NPAD // NS), NPAD // NS)])
    pltpu.sync_copy(ones_hbm, ones_v)

    def r_src(i):
        return row_hbm.at[pl.ds(w * EPW + i * K, K)]

    def issue_idx(i, b):
        pltpu.async_copy(r_src(i), ridxs[b], isems[b])

    def wait_idx(i, b):
        pltpu.make_async_copy(r_src(i), ridxs[b], isems[b]).wait()

    def issue_scatter(i, b):
        pltpu.async_copy(ones_v, deg_s.at[ridxs[b]], ssems[b], add=True)

    def wait_scatter(i, b):
        pltpu.make_async_copy(ones_v, deg_s.at[ridxs[b]], ssems[b]).wait()

    plsc.subcore_barrier()
    for b in range(2):
        issue_idx(b, b)

    def body(j, _):
        for b in range(2):
            i = 2 * j + b

            @pl.when(i < NCH)
            def _():
                wait_idx(i, b)
                issue_scatter(i, b)

        for b in range(2):
            i = 2 * j + b

            @pl.when(i + 2 < NCH)
            def _():
                wait_scatter(i, b)
                issue_idx(i + 2, b)

        return 0

    lax.fori_loop(0, (NCH + 1) // 2, body, 0)
    for i in range(NCH - 2, NCH):
        wait_scatter(i, i % 2)
    plsc.subcore_barrier()

    pltpu.sync_copy(deg_s.at[pl.ds(s * (NPAD // NS), NPAD // NS)],
                    out_hbm.at[c, pl.ds(s * (NPAD // NS), NPAD // NS)])


_hist_call = pl.kernel(
    _hist_body,
    out_type=jax.ShapeDtypeStruct((NC, NPAD), jnp.float32),
    mesh=_mesh,
    scratch_types=[
        pltpu.VMEM((K,), jnp.int32),
        pltpu.VMEM((K,), jnp.int32),
        pltpu.VMEM((K,), jnp.float32),
        pltpu.VMEM_SHARED((NPAD,), jnp.float32),
        pltpu.SemaphoreType.DMA,
        pltpu.SemaphoreType.DMA,
        pltpu.SemaphoreType.DMA,
        pltpu.SemaphoreType.DMA,
    ],
)


def _hist(row):
    ones = jnp.ones((K,), jnp.float32)
    zeros = jnp.zeros((NPAD // NS,), jnp.float32)
    return _hist_call(row, ones, zeros)


NBUF = 4


def _agg_body(g_hbm, col_hbm, row_hbm, zeros_hbm, out_hbm, acc_s, cidx0, cidx1,
              cidx2, cidx3, ridx0, ridx1, ridx2, ridx3, buf0, buf1, buf2, buf3,
              is0, is1, is2, is3, gs0, gs1, gs2, gs3, ss0, ss1, ss2, ss3):
    cidxs = (cidx0, cidx1, cidx2, cidx3)
    ridxs = (ridx0, ridx1, ridx2, ridx3)
    bufs = (buf0, buf1, buf2, buf3)
    isems = (is0, is1, is2, is3)
    gsems = (gs0, gs1, gs2, gs3)
    ssems = (ss0, ss1, ss2, ss3)
    c = lax.axis_index("c")
    s = lax.axis_index("s")
    w = c * NS + s

    pltpu.sync_copy(zeros_hbm, acc_s.at[pl.ds(s * APT, APT)])

    def c_src(i):
        return col_hbm.at[pl.ds(w * EPW + i * K, K)]

    def r_src(i):
        return row_hbm.at[pl.ds(w * EPW + i * K, K)]

    def issue_idx(i, b):
        pltpu.async_copy(c_src(i), cidxs[b], isems[b])
        pltpu.async_copy(r_src(i), ridxs[b], isems[b])

    def wait_idx(i, b):
        pltpu.make_async_copy(c_src(i), cidxs[b], isems[b]).wait()
        pltpu.make_async_copy(r_src(i), ridxs[b], isems[b]).wait()

    def issue_gather(i, b):
        pltpu.async_copy(g_hbm.at[cidxs[b]], bufs[b], gsems[b])

    def wait_gather(i, b):
        pltpu.make_async_copy(g_hbm.at[cidxs[b]], bufs[b], gsems[b]).wait()

    def issue_scatter(i, b):
        pltpu.async_copy(bufs[b], acc_s.at[ridxs[b]], ssems[b], add=True)

    def wait_scatter(i, b):
        pltpu.make_async_copy(bufs[b], acc_s.at[ridxs[b]], ssems[b]).wait()

    plsc.subcore_barrier()
    for b in range(NBUF):
        issue_idx(b, b)
    for b in range(NBUF):
        wait_idx(b, b)
        issue_gather(b, b)

    def body(j, _):
        for b in range(NBUF):
            i = NBUF * j + b

            @pl.when(i < NCH)
            def _():
                wait_gather(i, b)
                issue_scatter(i, b)

        for b in range(NBUF):
            i = NBUF * j + b

            @pl.when(i + NBUF < NCH)
            def _():
                wait_scatter(i, b)
                issue_idx(i + NBUF, b)

        for b in range(NBUF):
            i = NBUF * j + b

            @pl.when(i + NBUF < NCH)
            def _():
                wait_idx(i + NBUF, b)
                issue_gather(i + NBUF, b)

        return 0

    lax.fori_loop(0, (NCH + NBUF - 1) // NBUF, body, 0)
    for i in range(NCH - NBUF, NCH):
        wait_scatter(i, i % NBUF)
    plsc.subcore_barrier()

    pltpu.sync_copy(acc_s.at[pl.ds(s * APT, APT)],
                    out_hbm.at[c, pl.ds(s * APT, APT)])


_agg_call = pl.kernel(
    _agg_body,
    out_type=jax.ShapeDtypeStruct((NC, NPAD, D), jnp.float32),
    mesh=_mesh,
    scratch_types=[
        pltpu.VMEM_SHARED((NPAD, D), jnp.float32),
        pltpu.VMEM((K,), jnp.int32),
        pltpu.VMEM((K,), jnp.int32),
        pltpu.VMEM((K,), jnp.int32),
        pltpu.VMEM((K,), jnp.int32),
        pltpu.VMEM((K,), jnp.int32),
        pltpu.VMEM((K,), jnp.int32),
        pltpu.VMEM((K,), jnp.int32),
        pltpu.VMEM((K,), jnp.int32),
        pltpu.VMEM((K, D), jnp.float32),
        pltpu.VMEM((K, D), jnp.float32),
        pltpu.VMEM((K, D), jnp.float32),
        pltpu.VMEM((K, D), jnp.float32),
    ] + [pltpu.SemaphoreType.DMA] * 12,
)


def _aggregate(g, row, col):
    zeros = jnp.zeros((APT, D), jnp.float32)
    return _agg_call(g, col, row, zeros)


_RB = 2000


def _linear_body(x_ref, w_ref, degp_ref, g_ref, dis_ref):
    deg = degp_ref[0] + degp_ref[1] + 2.0
    dis = lax.rsqrt(deg)
    h = jnp.dot(x_ref[...], w_ref[...], preferred_element_type=jnp.float32)
    g_ref[...] = dis * h
    dis_ref[...] = dis


def _linear(x, weight, deg_part):
    return pl.pallas_call(
        _linear_body,
        grid=(N // _RB,),
        in_specs=[
            pl.BlockSpec((_RB, D), lambda i: (i, 0)),
            pl.BlockSpec((D, D), lambda i: (0, 0)),
            pl.BlockSpec((NC, _RB, 1), lambda i: (0, i, 0)),
        ],
        out_specs=[
            pl.BlockSpec((_RB, D), lambda i: (i, 0)),
            pl.BlockSpec((_RB, 1), lambda i: (i, 0)),
        ],
        out_shape=[
            jax.ShapeDtypeStruct((N, D), jnp.float32),
            jax.ShapeDtypeStruct((N, 1), jnp.float32),
        ],
    )(x, weight, deg_part.reshape(NC, NPAD, 1))


def _finish_body(acc_ref, g_ref, dis_ref, o_ref):
    acc = acc_ref[0] + acc_ref[1]
    o_ref[...] = jnp.maximum(dis_ref[...] * (acc + 2.0 * g_ref[...]), 0.0)


def _finish(acc, g, dis):
    return pl.pallas_call(
        _finish_body,
        grid=(N // _RB,),
        in_specs=[
            pl.BlockSpec((NC, _RB, D), lambda i: (0, i, 0)),
            pl.BlockSpec((_RB, D), lambda i: (i, 0)),
            pl.BlockSpec((_RB, 1), lambda i: (i, 0)),
        ],
        out_specs=pl.BlockSpec((_RB, D), lambda i: (i, 0)),
        out_shape=jax.ShapeDtypeStruct((N, D), jnp.float32),
    )(acc, g, dis)


def kernel(x, edge_index, weight):
    row = edge_index[0]
    col = edge_index[1]
    deg_part = _hist(row)
    g, dis = _linear(x, weight, deg_part)
    acc = _aggregate(g, row, col)
    out = _finish(acc, g, dis)
    return out


# speedup vs baseline: 1.0724x; 1.0724x over previous
import jax
import jax.numpy as jnp
from jax import lax
from jax.experimental import pallas as pl
from jax.experimental.pallas import tpu as pltpu
from jax.experimental.pallas import tpu_sc as plsc

N = 10000
E = 320000
D = 128

NC, NS = 2, 16
NW = NC * NS
NPAD = 10240
EPW = E // NW
K = 40
NCH = EPW // K
ROWS_PT = NPAD // NS
APT = NPAD // NS

_mesh = plsc.VectorSubcoreMesh(
    core_axis_name="c", subcore_axis_name="s", num_cores=NC, num_subcores=NS)


ASLOT = 8


def _hist_body(row_hbm, ones_hbm, zeros_hbm, out_hbm, *scr):
    ridxs = scr[0:ASLOT]
    ones_v = scr[ASLOT]
    deg_s = scr[ASLOT + 1]
    isems = scr[ASLOT + 2:2 * ASLOT + 2]
    ssems = scr[2 * ASLOT + 2:3 * ASLOT + 2]
    c = lax.axis_index("c")
    s = lax.axis_index("s")
    w = c * NS + s

    pltpu.sync_copy(zeros_hbm, deg_s.at[pl.ds(s * (NPAD // NS), NPAD // NS)])
    pltpu.sync_copy(ones_hbm, ones_v)

    def r_src(i):
        return row_hbm.at[pl.ds(w * EPW + i * K, K)]

    def issue_idx(i, b):
        pltpu.async_copy(r_src(i), ridxs[b], isems[b])

    def wait_idx(i, b):
        pltpu.make_async_copy(r_src(i), ridxs[b], isems[b]).wait()

    def issue_scatter(i, b):
        pltpu.async_copy(ones_v, deg_s.at[ridxs[b]], ssems[b], add=True)

    def wait_scatter(i, b):
        pltpu.make_async_copy(ones_v, deg_s.at[ridxs[b]], ssems[b]).wait()

    plsc.subcore_barrier()
    for b in range(ASLOT):
        issue_idx(b, b)

    def body(j, _):
        for b in range(ASLOT):
            i = ASLOT * j + b

            @pl.when(i < NCH)
            def _():
                wait_idx(i, b)
                issue_scatter(i, b)

        for b in range(ASLOT):
            i = ASLOT * j + b

            @pl.when(i + ASLOT < NCH)
            def _():
                wait_scatter(i, b)
                issue_idx(i + ASLOT, b)

        return 0

    lax.fori_loop(0, (NCH + ASLOT - 1) // ASLOT, body, 0)
    for i in range(NCH - ASLOT, NCH):
        wait_scatter(i, i % ASLOT)
    plsc.subcore_barrier()

    pltpu.sync_copy(deg_s.at[pl.ds(s * (NPAD // NS), NPAD // NS)],
                    out_hbm.at[c, pl.ds(s * (NPAD // NS), NPAD // NS)])


_hist_call = pl.kernel(
    _hist_body,
    out_type=jax.ShapeDtypeStruct((NC, NPAD), jnp.float32),
    mesh=_mesh,
    scratch_types=(
        [pltpu.VMEM((K,), jnp.int32) for _ in range(ASLOT)]
        + [pltpu.VMEM((K,), jnp.float32),
           pltpu.VMEM_SHARED((NPAD,), jnp.float32)]
        + [pltpu.SemaphoreType.DMA] * (2 * ASLOT)
    ),
)


def _hist(row):
    ones = jnp.ones((K,), jnp.float32)
    zeros = jnp.zeros((NPAD // NS,), jnp.float32)
    return _hist_call(row, ones, zeros)


NBUF = 6


def _agg_body(g_hbm, col_hbm, row_hbm, zeros_hbm, out_hbm, acc_s, *scr):
    cidxs = scr[0:NBUF]
    ridxs = scr[NBUF:2 * NBUF]
    bufs = scr[2 * NBUF:3 * NBUF]
    isems = scr[3 * NBUF:4 * NBUF]
    gsems = scr[4 * NBUF:5 * NBUF]
    ssems = scr[5 * NBUF:6 * NBUF]
    c = lax.axis_index("c")
    s = lax.axis_index("s")
    w = c * NS + s

    pltpu.sync_copy(zeros_hbm, acc_s.at[pl.ds(s * APT, APT)])

    def c_src(i):
        return col_hbm.at[pl.ds(w * EPW + i * K, K)]

    def r_src(i):
        return row_hbm.at[pl.ds(w * EPW + i * K, K)]

    def issue_idx(i, b):
        pltpu.async_copy(c_src(i), cidxs[b], isems[b])
        pltpu.async_copy(r_src(i), ridxs[b], isems[b])

    def wait_idx(i, b):
        pltpu.make_async_copy(c_src(i), cidxs[b], isems[b]).wait()
        pltpu.make_async_copy(r_src(i), ridxs[b], isems[b]).wait()

    def issue_gather(i, b):
        pltpu.async_copy(g_hbm.at[cidxs[b]], bufs[b], gsems[b])

    def wait_gather(i, b):
        pltpu.make_async_copy(g_hbm.at[cidxs[b]], bufs[b], gsems[b]).wait()

    def issue_scatter(i, b):
        pltpu.async_copy(bufs[b], acc_s.at[ridxs[b]], ssems[b], add=True)

    def wait_scatter(i, b):
        pltpu.make_async_copy(bufs[b], acc_s.at[ridxs[b]], ssems[b]).wait()

    plsc.subcore_barrier()
    for b in range(NBUF):
        issue_idx(b, b)
    for b in range(NBUF):
        wait_idx(b, b)
        issue_gather(b, b)

    def body(j, _):
        for b in range(NBUF):
            i = NBUF * j + b

            @pl.when(i < NCH)
            def _():
                wait_gather(i, b)
                issue_scatter(i, b)

        for b in range(NBUF):
            i = NBUF * j + b

            @pl.when(i + NBUF < NCH)
            def _():
                wait_scatter(i, b)
                issue_idx(i + NBUF, b)

        for b in range(NBUF):
            i = NBUF * j + b

            @pl.when(i + NBUF < NCH)
            def _():
                wait_idx(i + NBUF, b)
                issue_gather(i + NBUF, b)

        return 0

    lax.fori_loop(0, (NCH + NBUF - 1) // NBUF, body, 0)
    for i in range(NCH - NBUF, NCH):
        wait_scatter(i, i % NBUF)
    plsc.subcore_barrier()

    pltpu.sync_copy(acc_s.at[pl.ds(s * APT, APT)],
                    out_hbm.at[c, pl.ds(s * APT, APT)])


_agg_call = pl.kernel(
    _agg_body,
    out_type=jax.ShapeDtypeStruct((NC, NPAD, D), jnp.float32),
    mesh=_mesh,
    scratch_types=(
        [pltpu.VMEM_SHARED((NPAD, D), jnp.float32)]
        + [pltpu.VMEM((K,), jnp.int32) for _ in range(2 * NBUF)]
        + [pltpu.VMEM((K, D), jnp.float32) for _ in range(NBUF)]
        + [pltpu.SemaphoreType.DMA] * (3 * NBUF)
    ),
)


def _aggregate(g, row, col):
    zeros = jnp.zeros((APT, D), jnp.float32)
    return _agg_call(g, col, row, zeros)


_RB = 2000


def _linear_body(x_ref, w_ref, degp_ref, g_ref, dis_ref):
    deg = degp_ref[0] + degp_ref[1] + 2.0
    dis = lax.rsqrt(deg)
    h = jnp.dot(x_ref[...], w_ref[...], preferred_element_type=jnp.float32)
    g_ref[...] = dis * h
    dis_ref[...] = dis


def _linear(x, weight, deg_part):
    return pl.pallas_call(
        _linear_body,
        grid=(N // _RB,),
        in_specs=[
            pl.BlockSpec((_RB, D), lambda i: (i, 0)),
            pl.BlockSpec((D, D), lambda i: (0, 0)),
            pl.BlockSpec((NC, _RB, 1), lambda i: (0, i, 0)),
        ],
        out_specs=[
            pl.BlockSpec((_RB, D), lambda i: (i, 0)),
            pl.BlockSpec((_RB, 1), lambda i: (i, 0)),
        ],
        out_shape=[
            jax.ShapeDtypeStruct((N, D), jnp.float32),
            jax.ShapeDtypeStruct((N, 1), jnp.float32),
        ],
    )(x, weight, deg_part.reshape(NC, NPAD, 1))


def _finish_body(acc_ref, g_ref, dis_ref, o_ref):
    acc = acc_ref[0] + acc_ref[1]
    o_ref[...] = jnp.maximum(dis_ref[...] * (acc + 2.0 * g_ref[...]), 0.0)


def _finish(acc, g, dis):
    return pl.pallas_call(
        _finish_body,
        grid=(N // _RB,),
        in_specs=[
            pl.BlockSpec((NC, _RB, D), lambda i: (0, i, 0)),
            pl.BlockSpec((_RB, D), lambda i: (i, 0)),
            pl.BlockSpec((_RB, 1), lambda i: (i, 0)),
        ],
        out_specs=pl.BlockSpec((_RB, D), lambda i: (i, 0)),
        out_shape=jax.ShapeDtypeStruct((N, D), jnp.float32),
    )(acc, g, dis)


def kernel(x, edge_index, weight):
    row = edge_index[0]
    col = edge_index[1]
    deg_part = _hist(row)
    g, dis = _linear(x, weight, deg_part)
    acc = _aggregate(g, row, col)
    out = _finish(acc, g, dis)
    return out


# speedup vs baseline: 1.1013x; 1.0269x over previous
import jax
import jax.numpy as jnp
from jax import lax
from jax.experimental import pallas as pl
from jax.experimental.pallas import tpu as pltpu
from jax.experimental.pallas import tpu_sc as plsc

N = 10000
E = 320000
D = 128

NC, NS = 2, 16
NW = NC * NS
NPAD = 10240
EPW = E // NW
K = 40
NCH = EPW // K
ROWS_PT = NPAD // NS
APT = NPAD // NS

_mesh = plsc.VectorSubcoreMesh(
    core_axis_name="c", subcore_axis_name="s", num_cores=NC, num_subcores=NS)


ASLOT = 8


def _hist_body(row_hbm, ones_hbm, zeros_hbm, out_hbm, *scr):
    ridxs = scr[0:ASLOT]
    ones_v = scr[ASLOT]
    deg_s = scr[ASLOT + 1]
    isems = scr[ASLOT + 2:2 * ASLOT + 2]
    ssems = scr[2 * ASLOT + 2:3 * ASLOT + 2]
    c = lax.axis_index("c")
    s = lax.axis_index("s")
    w = c * NS + s

    pltpu.sync_copy(zeros_hbm, deg_s.at[pl.ds(s * (NPAD // NS), NPAD // NS)])
    pltpu.sync_copy(ones_hbm, ones_v)

    def r_src(i):
        return row_hbm.at[pl.ds(w * EPW + i * K, K)]

    def issue_idx(i, b):
        pltpu.async_copy(r_src(i), ridxs[b], isems[b])

    def wait_idx(i, b):
        pltpu.make_async_copy(r_src(i), ridxs[b], isems[b]).wait()

    def issue_scatter(i, b):
        pltpu.async_copy(ones_v, deg_s.at[ridxs[b]], ssems[b], add=True)

    def wait_scatter(i, b):
        pltpu.make_async_copy(ones_v, deg_s.at[ridxs[b]], ssems[b]).wait()

    plsc.subcore_barrier()
    for b in range(ASLOT):
        issue_idx(b, b)

    def body(j, _):
        for b in range(ASLOT):
            i = ASLOT * j + b

            @pl.when(i < NCH)
            def _():
                wait_idx(i, b)
                issue_scatter(i, b)

        for b in range(ASLOT):
            i = ASLOT * j + b

            @pl.when(i + ASLOT < NCH)
            def _():
                wait_scatter(i, b)
                issue_idx(i + ASLOT, b)

        return 0

    lax.fori_loop(0, (NCH + ASLOT - 1) // ASLOT, body, 0)
    for i in range(NCH - ASLOT, NCH):
        wait_scatter(i, i % ASLOT)
    plsc.subcore_barrier()

    pltpu.sync_copy(deg_s.at[pl.ds(s * (NPAD // NS), NPAD // NS)],
                    out_hbm.at[c, pl.ds(s * (NPAD // NS), NPAD // NS)])


_hist_call = pl.kernel(
    _hist_body,
    out_type=jax.ShapeDtypeStruct((NC, NPAD), jnp.float32),
    mesh=_mesh,
    scratch_types=(
        [pltpu.VMEM((K,), jnp.int32) for _ in range(ASLOT)]
        + [pltpu.VMEM((K,), jnp.float32),
           pltpu.VMEM_SHARED((NPAD,), jnp.float32)]
        + [pltpu.SemaphoreType.DMA] * (2 * ASLOT)
    ),
)


def _hist(row):
    ones = jnp.ones((K,), jnp.float32)
    zeros = jnp.zeros((NPAD // NS,), jnp.float32)
    return _hist_call(row, ones, zeros)


NBUF = 8


def _agg_body(g_hbm, col_hbm, row_hbm, zeros_hbm, out_hbm, acc_s, *scr):
    cidxs = scr[0:NBUF]
    ridxs = scr[NBUF:2 * NBUF]
    bufs = scr[2 * NBUF:3 * NBUF]
    isems = scr[3 * NBUF:4 * NBUF]
    gsems = scr[4 * NBUF:5 * NBUF]
    ssems = scr[5 * NBUF:6 * NBUF]
    c = lax.axis_index("c")
    s = lax.axis_index("s")
    w = c * NS + s

    pltpu.sync_copy(zeros_hbm, acc_s.at[pl.ds(s * APT, APT)])

    def c_src(i):
        return col_hbm.at[pl.ds(w * EPW + i * K, K)]

    def r_src(i):
        return row_hbm.at[pl.ds(w * EPW + i * K, K)]

    def issue_idx(i, b):
        pltpu.async_copy(c_src(i), cidxs[b], isems[b])
        pltpu.async_copy(r_src(i), ridxs[b], isems[b])

    def wait_idx(i, b):
        pltpu.make_async_copy(c_src(i), cidxs[b], isems[b]).wait()
        pltpu.make_async_copy(r_src(i), ridxs[b], isems[b]).wait()

    def issue_gather(i, b):
        pltpu.async_copy(g_hbm.at[cidxs[b]], bufs[b], gsems[b])

    def wait_gather(i, b):
        pltpu.make_async_copy(g_hbm.at[cidxs[b]], bufs[b], gsems[b]).wait()

    def issue_scatter(i, b):
        pltpu.async_copy(bufs[b], acc_s.at[ridxs[b]], ssems[b], add=True)

    def wait_scatter(i, b):
        pltpu.make_async_copy(bufs[b], acc_s.at[ridxs[b]], ssems[b]).wait()

    plsc.subcore_barrier()
    for b in range(NBUF):
        issue_idx(b, b)
    for b in range(NBUF):
        wait_idx(b, b)
        issue_gather(b, b)

    def body(j, _):
        for b in range(NBUF):
            i = NBUF * j + b

            @pl.when(i < NCH)
            def _():
                wait_gather(i, b)
                issue_scatter(i, b)

        for b in range(NBUF):
            i = NBUF * j + b

            @pl.when(i + NBUF < NCH)
            def _():
                wait_scatter(i, b)
                issue_idx(i + NBUF, b)

        for b in range(NBUF):
            i = NBUF * j + b

            @pl.when(i + NBUF < NCH)
            def _():
                wait_idx(i + NBUF, b)
                issue_gather(i + NBUF, b)

        return 0

    lax.fori_loop(0, (NCH + NBUF - 1) // NBUF, body, 0)
    for i in range(NCH - NBUF, NCH):
        wait_scatter(i, i % NBUF)
    plsc.subcore_barrier()

    pltpu.sync_copy(acc_s.at[pl.ds(s * APT, APT)],
                    out_hbm.at[c, pl.ds(s * APT, APT)])


_agg_call = pl.kernel(
    _agg_body,
    out_type=jax.ShapeDtypeStruct((NC, NPAD, D), jnp.float32),
    mesh=_mesh,
    scratch_types=(
        [pltpu.VMEM_SHARED((NPAD, D), jnp.float32)]
        + [pltpu.VMEM((K,), jnp.int32) for _ in range(2 * NBUF)]
        + [pltpu.VMEM((K, D), jnp.float32) for _ in range(NBUF)]
        + [pltpu.SemaphoreType.DMA] * (3 * NBUF)
    ),
)


def _aggregate(g, row, col):
    zeros = jnp.zeros((APT, D), jnp.float32)
    return _agg_call(g, col, row, zeros)


_RB = 2000


def _linear_body(x_ref, w_ref, degp_ref, g_ref, dis_ref):
    deg = degp_ref[0] + degp_ref[1] + 2.0
    dis = lax.rsqrt(deg)
    h = jnp.dot(x_ref[...], w_ref[...], preferred_element_type=jnp.float32)
    g_ref[...] = dis * h
    dis_ref[...] = dis


def _linear(x, weight, deg_part):
    return pl.pallas_call(
        _linear_body,
        grid=(N // _RB,),
        in_specs=[
            pl.BlockSpec((_RB, D), lambda i: (i, 0)),
            pl.BlockSpec((D, D), lambda i: (0, 0)),
            pl.BlockSpec((NC, _RB, 1), lambda i: (0, i, 0)),
        ],
        out_specs=[
            pl.BlockSpec((_RB, D), lambda i: (i, 0)),
            pl.BlockSpec((_RB, 1), lambda i: (i, 0)),
        ],
        out_shape=[
            jax.ShapeDtypeStruct((N, D), jnp.float32),
            jax.ShapeDtypeStruct((N, 1), jnp.float32),
        ],
    )(x, weight, deg_part.reshape(NC, NPAD, 1))


def _finish_body(acc_ref, g_ref, dis_ref, o_ref):
    acc = acc_ref[0] + acc_ref[1]
    o_ref[...] = jnp.maximum(dis_ref[...] * (acc + 2.0 * g_ref[...]), 0.0)


def _finish(acc, g, dis):
    return pl.pallas_call(
        _finish_body,
        grid=(N // _RB,),
        in_specs=[
            pl.BlockSpec((NC, _RB, D), lambda i: (0, i, 0)),
            pl.BlockSpec((_RB, D), lambda i: (i, 0)),
            pl.BlockSpec((_RB, 1), lambda i: (i, 0)),
        ],
        out_specs=pl.BlockSpec((_RB, D), lambda i: (i, 0)),
        out_shape=jax.ShapeDtypeStruct((N, D), jnp.float32),
    )(acc, g, dis)


def kernel(x, edge_index, weight):
    row = edge_index[0]
    col = edge_index[1]
    deg_part = _hist(row)
    g, dis = _linear(x, weight, deg_part)
    acc = _aggregate(g, row, col)
    out = _finish(acc, g, dis)
    return out


# speedup vs baseline: 1.1430x; 1.0378x over previous
import jax
import jax.numpy as jnp
from jax import lax
from jax.experimental import pallas as pl
from jax.experimental.pallas import tpu as pltpu
from jax.experimental.pallas import tpu_sc as plsc

N = 10000
E = 320000
D = 128

NC, NS = 2, 16
NW = NC * NS
NPAD = 10240
EPW = E // NW
K = 40
NCH = EPW // K
KA = 80
NCHA = EPW // KA
ROWS_PT = NPAD // NS
APT = NPAD // NS

_mesh = plsc.VectorSubcoreMesh(
    core_axis_name="c", subcore_axis_name="s", num_cores=NC, num_subcores=NS)


ASLOT = 8


def _hist_body(row_hbm, ones_hbm, zeros_hbm, out_hbm, *scr):
    ridxs = scr[0:ASLOT]
    ones_v = scr[ASLOT]
    deg_s = scr[ASLOT + 1]
    isems = scr[ASLOT + 2:2 * ASLOT + 2]
    ssems = scr[2 * ASLOT + 2:3 * ASLOT + 2]
    c = lax.axis_index("c")
    s = lax.axis_index("s")
    w = c * NS + s

    pltpu.sync_copy(zeros_hbm, deg_s.at[pl.ds(s * (NPAD // NS), NPAD // NS)])
    pltpu.sync_copy(ones_hbm, ones_v)

    def r_src(i):
        return row_hbm.at[pl.ds(w * EPW + i * KA, KA)]

    def issue_idx(i, b):
        pltpu.async_copy(r_src(i), ridxs[b], isems[b])

    def wait_idx(i, b):
        pltpu.make_async_copy(r_src(i), ridxs[b], isems[b]).wait()

    def issue_scatter(i, b):
        pltpu.async_copy(ones_v, deg_s.at[ridxs[b]], ssems[b], add=True)

    def wait_scatter(i, b):
        pltpu.make_async_copy(ones_v, deg_s.at[ridxs[b]], ssems[b]).wait()

    plsc.subcore_barrier()
    for b in range(ASLOT):
        issue_idx(b, b)

    def body(j, _):
        for b in range(ASLOT):
            i = ASLOT * j + b

            @pl.when(i < NCHA)
            def _():
                wait_idx(i, b)
                issue_scatter(i, b)

        for b in range(ASLOT):
            i = ASLOT * j + b

            @pl.when(i + ASLOT < NCHA)
            def _():
                wait_scatter(i, b)
                issue_idx(i + ASLOT, b)

        return 0

    lax.fori_loop(0, (NCHA + ASLOT - 1) // ASLOT, body, 0)
    for i in range(NCHA - ASLOT, NCHA):
        wait_scatter(i, i % ASLOT)
    plsc.subcore_barrier()

    pltpu.sync_copy(deg_s.at[pl.ds(s * (NPAD // NS), NPAD // NS)],
                    out_hbm.at[c, pl.ds(s * (NPAD // NS), NPAD // NS)])


_hist_call = pl.kernel(
    _hist_body,
    out_type=jax.ShapeDtypeStruct((NC, NPAD), jnp.float32),
    mesh=_mesh,
    scratch_types=(
        [pltpu.VMEM((KA,), jnp.int32) for _ in range(ASLOT)]
        + [pltpu.VMEM((KA,), jnp.float32),
           pltpu.VMEM_SHARED((NPAD,), jnp.float32)]
        + [pltpu.SemaphoreType.DMA] * (2 * ASLOT)
    ),
)


def _hist(row):
    ones = jnp.ones((KA,), jnp.float32)
    zeros = jnp.zeros((NPAD // NS,), jnp.float32)
    return _hist_call(row, ones, zeros)


NBUF = 8


def _agg_body(g_hbm, col_hbm, row_hbm, zeros_hbm, out_hbm, acc_s, *scr):
    cidxs = scr[0:NBUF]
    ridxs = scr[NBUF:2 * NBUF]
    bufs = scr[2 * NBUF:3 * NBUF]
    isems = scr[3 * NBUF:4 * NBUF]
    gsems = scr[4 * NBUF:5 * NBUF]
    ssems = scr[5 * NBUF:6 * NBUF]
    c = lax.axis_index("c")
    s = lax.axis_index("s")
    w = c * NS + s

    pltpu.sync_copy(zeros_hbm, acc_s.at[pl.ds(s * APT, APT)])

    def c_src(i):
        return col_hbm.at[pl.ds(w * EPW + i * K, K)]

    def r_src(i):
        return row_hbm.at[pl.ds(w * EPW + i * K, K)]

    def issue_idx(i, b):
        pltpu.async_copy(c_src(i), cidxs[b], isems[b])
        pltpu.async_copy(r_src(i), ridxs[b], isems[b])

    def wait_idx(i, b):
        pltpu.make_async_copy(c_src(i), cidxs[b], isems[b]).wait()
        pltpu.make_async_copy(r_src(i), ridxs[b], isems[b]).wait()

    def issue_gather(i, b):
        pltpu.async_copy(g_hbm.at[cidxs[b]], bufs[b], gsems[b])

    def wait_gather(i, b):
        pltpu.make_async_copy(g_hbm.at[cidxs[b]], bufs[b], gsems[b]).wait()

    def issue_scatter(i, b):
        pltpu.async_copy(bufs[b], acc_s.at[ridxs[b]], ssems[b], add=True)

    def wait_scatter(i, b):
        pltpu.make_async_copy(bufs[b], acc_s.at[ridxs[b]], ssems[b]).wait()

    plsc.subcore_barrier()
    for b in range(NBUF):
        issue_idx(b, b)
    for b in range(NBUF):
        wait_idx(b, b)
        issue_gather(b, b)

    def body(j, _):
        for b in range(NBUF):
            i = NBUF * j + b

            @pl.when(i < NCH)
            def _():
                wait_gather(i, b)
                issue_scatter(i, b)

        for b in range(NBUF):
            i = NBUF * j + b

            @pl.when(i + NBUF < NCH)
            def _():
                wait_scatter(i, b)
                issue_idx(i + NBUF, b)

        for b in range(NBUF):
            i = NBUF * j + b

            @pl.when(i + NBUF < NCH)
            def _():
                wait_idx(i + NBUF, b)
                issue_gather(i + NBUF, b)

        return 0

    lax.fori_loop(0, (NCH + NBUF - 1) // NBUF, body, 0)
    for i in range(NCH - NBUF, NCH):
        wait_scatter(i, i % NBUF)
    plsc.subcore_barrier()

    pltpu.sync_copy(acc_s.at[pl.ds(s * APT, APT)],
                    out_hbm.at[c, pl.ds(s * APT, APT)])


_agg_call = pl.kernel(
    _agg_body,
    out_type=jax.ShapeDtypeStruct((NC, NPAD, D), jnp.float32),
    mesh=_mesh,
    scratch_types=(
        [pltpu.VMEM_SHARED((NPAD, D), jnp.float32)]
        + [pltpu.VMEM((K,), jnp.int32) for _ in range(2 * NBUF)]
        + [pltpu.VMEM((K, D), jnp.float32) for _ in range(NBUF)]
        + [pltpu.SemaphoreType.DMA] * (3 * NBUF)
    ),
)


def _aggregate(g, row, col):
    zeros = jnp.zeros((APT, D), jnp.float32)
    return _agg_call(g, col, row, zeros)


_RB = 2000


def _linear_body(x_ref, w_ref, degp_ref, g_ref, dis_ref):
    deg = degp_ref[0] + degp_ref[1] + 2.0
    dis = lax.rsqrt(deg)
    h = jnp.dot(x_ref[...], w_ref[...], preferred_element_type=jnp.float32)
    g_ref[...] = dis * h
    dis_ref[...] = dis


def _linear(x, weight, deg_part):
    return pl.pallas_call(
        _linear_body,
        grid=(N // _RB,),
        in_specs=[
            pl.BlockSpec((_RB, D), lambda i: (i, 0)),
            pl.BlockSpec((D, D), lambda i: (0, 0)),
            pl.BlockSpec((NC, _RB, 1), lambda i: (0, i, 0)),
        ],
        out_specs=[
            pl.BlockSpec((_RB, D), lambda i: (i, 0)),
            pl.BlockSpec((_RB, 1), lambda i: (i, 0)),
        ],
        out_shape=[
            jax.ShapeDtypeStruct((N, D), jnp.float32),
            jax.ShapeDtypeStruct((N, 1), jnp.float32),
        ],
    )(x, weight, deg_part.reshape(NC, NPAD, 1))


def _finish_body(acc_ref, g_ref, dis_ref, o_ref):
    acc = acc_ref[0] + acc_ref[1]
    o_ref[...] = jnp.maximum(dis_ref[...] * (acc + 2.0 * g_ref[...]), 0.0)


def _finish(acc, g, dis):
    return pl.pallas_call(
        _finish_body,
        grid=(N // _RB,),
        in_specs=[
            pl.BlockSpec((NC, _RB, D), lambda i: (0, i, 0)),
            pl.BlockSpec((_RB, D), lambda i: (i, 0)),
            pl.BlockSpec((_RB, 1), lambda i: (i, 0)),
        ],
        out_specs=pl.BlockSpec((_RB, D), lambda i: (i, 0)),
        out_shape=jax.ShapeDtypeStruct((N, D), jnp.float32),
    )(acc, g, dis)


def kernel(x, edge_index, weight):
    row = edge_index[0]
    col = edge_index[1]
    deg_part = _hist(row)
    g, dis = _linear(x, weight, deg_part)
    acc = _aggregate(g, row, col)
    out = _finish(acc, g, dis)
    return out
